# baseline (speedup 1.0000x reference)
import jax
import jax.numpy as jnp
from jax.experimental import pallas as pl
from jax.experimental.pallas import tpu as pltpu

_DOT_DIMS = (((1,), (0,)), ((), ()))
_CK = 128
_NC = 3


def _dot(w_ref, x_refs):
    w = w_ref[...]
    acc = jax.lax.dot_general(
        w[:, :_CK], x_refs[0][0, 0], _DOT_DIMS,
        preferred_element_type=jnp.float32)
    for k in range(1, _NC):
        acc += jax.lax.dot_general(
            w[:, k * _CK:(k + 1) * _CK], x_refs[k][0, 0], _DOT_DIMS,
            preferred_element_type=jnp.float32)
    return acc


def _head_kernel(x0_ref, x1_ref, x2_ref, wc_ref, bc_ref, wr_ref, br_ref,
                 wd_ref, bd_ref, cls_ref, reg_ref, dir_ref):
    xs = (x0_ref, x1_ref, x2_ref)
    cls_ref[0] = _dot(wc_ref, xs) + bc_ref[...]
    reg_ref[0] = _dot(wr_ref, xs) + br_ref[...]
    dir_ref[0] = _dot(wd_ref, xs) + bd_ref[...]


def kernel(x, W_cls, b_cls, W_reg, b_reg, W_dir, b_dir):
    B, C, H, W = x.shape
    HW = H * W
    T = 5376
    G = pl.cdiv(HW, T)
    xr = x.reshape(B, _NC, _CK, HW)
    oc, og, od = W_cls.shape[0], W_reg.shape[0], W_dir.shape[0]
    bc = b_cls.reshape(oc, 1)
    bg = b_reg.reshape(og, 1)
    bd = b_dir.reshape(od, 1)

    def xspec(k):
        return pl.BlockSpec((1, 1, _CK, T), lambda b, j, k=k: (b, k, 0, j))

    def wspec(o):
        return pl.BlockSpec((o, C), lambda b, j: (0, 0))

    def bspec(o):
        return pl.BlockSpec((o, 1), lambda b, j: (0, 0))

    def ospec(o):
        return pl.BlockSpec((1, o, T), lambda b, j: (b, 0, j))

    outs = pl.pallas_call(
        _head_kernel,
        grid=(B, G),
        in_specs=[
            xspec(0), xspec(1), xspec(2),
            wspec(oc), bspec(oc), wspec(og), bspec(og), wspec(od), bspec(od),
        ],
        out_specs=[ospec(oc), ospec(og), ospec(od)],
        out_shape=[
            jax.ShapeDtypeStruct((B, oc, HW), x.dtype),
            jax.ShapeDtypeStruct((B, og, HW), x.dtype),
            jax.ShapeDtypeStruct((B, od, HW), x.dtype),
        ],
        compiler_params=pltpu.CompilerParams(
            dimension_semantics=("parallel", "parallel")),
    )(xr, xr, xr, W_cls, bc, W_reg, bg, W_dir, bd)
    cls_o, reg_o, dir_o = outs
    return (cls_o.reshape(B, oc, H, W),
            reg_o.reshape(B, og, H, W),
            dir_o.reshape(B, od, H, W))


# speedup vs baseline: 2.2060x; 2.2060x over previous
import jax
import jax.numpy as jnp
from jax.experimental import pallas as pl
from jax.experimental.pallas import tpu as pltpu

_DOT_DIMS = (((1,), (0,)), ((), ()))


def _head_kernel(x_ref, wc_ref, bc_ref, wr_ref, br_ref, wd_ref, bd_ref,
                 cls_ref, reg_ref, dir_ref):
    xb = x_ref[0]
    cls_ref[0] = jax.lax.dot_general(
        wc_ref[...], xb, _DOT_DIMS, preferred_element_type=jnp.float32) + bc_ref[...]
    reg_ref[0] = jax.lax.dot_general(
        wr_ref[...], xb, _DOT_DIMS, preferred_element_type=jnp.float32) + br_ref[...]
    dir_ref[0] = jax.lax.dot_general(
        wd_ref[...], xb, _DOT_DIMS, preferred_element_type=jnp.float32) + bd_ref[...]


def kernel(x, W_cls, b_cls, W_reg, b_reg, W_dir, b_dir):
    B, C, H, W = x.shape
    HW = H * W
    T = 13440
    G = pl.cdiv(HW, T)
    xf = x.reshape(B, C, HW)
    oc, og, od = W_cls.shape[0], W_reg.shape[0], W_dir.shape[0]
    bc = b_cls.reshape(oc, 1)
    bg = b_reg.reshape(og, 1)
    bd = b_dir.reshape(od, 1)

    def wspec(o):
        return pl.BlockSpec((o, C), lambda b, j: (0, 0))

    def bspec(o):
        return pl.BlockSpec((o, 1), lambda b, j: (0, 0))

    def ospec(o):
        return pl.BlockSpec((1, o, T), lambda b, j: (b, 0, j))

    outs = pl.pallas_call(
        _head_kernel,
        grid=(B, G),
        in_specs=[
            pl.BlockSpec((1, C, T), lambda b, j: (b, 0, j)),
            wspec(oc), bspec(oc), wspec(og), bspec(og), wspec(od), bspec(od),
        ],
        out_specs=[ospec(oc), ospec(og), ospec(od)],
        out_shape=[
            jax.ShapeDtypeStruct((B, oc, HW), x.dtype),
            jax.ShapeDtypeStruct((B, og, HW), x.dtype),
            jax.ShapeDtypeStruct((B, od, HW), x.dtype),
        ],
        compiler_params=pltpu.CompilerParams(
            dimension_semantics=("parallel", "parallel")),
    )(xf, W_cls, bc, W_reg, bg, W_dir, bd)
    cls_o, reg_o, dir_o = outs
    return (cls_o.reshape(B, oc, H, W),
            reg_o.reshape(B, og, H, W),
            dir_o.reshape(B, od, H, W))
